# baseline (speedup 1.0000x reference)
import functools

import jax
import jax.numpy as jnp
from jax import lax
from jax.experimental import pallas as pl
from jax.experimental.pallas import tpu as pltpu
from jax.experimental.pallas import tpu_sc as plsc

_NUM_KP = 6
_ROW = 768


_CH = 64


def _build_sc_call(B, NC, NS):
    NW = NC * NS
    b_per_w = B // NW
    CH = _CH
    n_ch = b_per_w // CH
    mesh = plsc.VectorSubcoreMesh(core_axis_name="c", subcore_axis_name="s")

    @functools.partial(
        pl.kernel,
        mesh=mesh,
        out_type=jax.ShapeDtypeStruct((B, _ROW), jnp.float32),
        scratch_types=[
            pltpu.VMEM((n_ch, CH), jnp.int32),
            pltpu.VMEM((2, CH, _ROW), jnp.float32),
            pltpu.SemaphoreType.DMA,
            pltpu.SemaphoreType.DMA,
            pltpu.SemaphoreType.DMA,
        ],
    )
    def sc_gather(table_hbm, idx_hbm, out_hbm, idx_v, rows_v, gsem, s0, s1):
        wid = lax.axis_index("s") * NC + lax.axis_index("c")
        base = wid * b_per_w
        pltpu.sync_copy(idx_hbm.at[wid], idx_v)
        ssem = (s0, s1)
        outstanding = [None, None]
        for c in range(n_ch):
            p = c & 1
            if outstanding[p] is not None:
                outstanding[p].wait()
            pltpu.async_copy(table_hbm.at[idx_v.at[c]], rows_v.at[p], gsem).wait()
            outstanding[p] = pltpu.async_copy(
                rows_v.at[p], out_hbm.at[pl.ds(base + c * CH, CH)], ssem[p])
        for p in range(2):
            if outstanding[p] is not None:
                outstanding[p].wait()

    return sc_gather


def kernel(grip_state, distinction_table, state_table):
    B = grip_state.shape[0]
    num_kp = distinction_table.shape[0]
    info = plsc.get_sparse_core_info()
    NC, NS = info.num_cores, info.num_subcores
    NW = NC * NS

    f = jnp.broadcast_to(distinction_table[None], (2,) + distinction_table.shape)
    s = jnp.broadcast_to(state_table[:, None, :], (2, num_kp, state_table.shape[-1]))
    fused = jnp.concatenate([f, s], axis=-1).reshape(2, _ROW)

    b_per_w = B // NW
    idx = grip_state.astype(jnp.int32).reshape(NW, b_per_w // _CH, _CH)

    out = _build_sc_call(B, NC, NS)(fused, idx)
    return out.reshape(B, num_kp, _ROW // num_kp)


# speedup vs baseline: 2.6779x; 2.6779x over previous
import functools

import jax
import jax.numpy as jnp
from jax import lax
from jax.experimental import pallas as pl
from jax.experimental.pallas import tpu as pltpu
from jax.experimental.pallas import tpu_sc as plsc

_NUM_KP = 6
_ROW = 768


_CH = 64


def _build_sc_call(B, NC, NS):
    NW = NC * NS
    b_per_w = B // NW
    CH = _CH
    n_ch = b_per_w // CH
    mesh = plsc.VectorSubcoreMesh(core_axis_name="c", subcore_axis_name="s")

    @functools.partial(
        pl.kernel,
        mesh=mesh,
        out_type=jax.ShapeDtypeStruct((B, _ROW), jnp.float32),
        scratch_types=[
            pltpu.VMEM((n_ch, CH), jnp.int32),
            pltpu.VMEM((2, CH, _ROW), jnp.float32),
            pltpu.SemaphoreType.DMA,
            pltpu.SemaphoreType.DMA,
            pltpu.SemaphoreType.DMA,
        ],
    )
    def sc_gather(table_hbm, idx_hbm, out_hbm, idx_v, rows_v, gsem, s0, s1):
        wid = lax.axis_index("s") * NC + lax.axis_index("c")
        base = wid * b_per_w
        pltpu.sync_copy(idx_hbm.at[wid], idx_v)
        ssem = (s0, s1)
        outstanding = [None, None]
        for c in range(n_ch):
            p = c & 1
            if outstanding[p] is not None:
                outstanding[p].wait()
            pltpu.async_copy(table_hbm.at[idx_v.at[c]], rows_v.at[p], gsem).wait()
            outstanding[p] = pltpu.async_copy(
                rows_v.at[p], out_hbm.at[pl.ds(base + c * CH, CH)], ssem[p])
        for p in range(2):
            if outstanding[p] is not None:
                outstanding[p].wait()

    return sc_gather


def kernel(grip_state, distinction_table, state_table):
    B = grip_state.shape[0]
    num_kp = distinction_table.shape[0]
    info = plsc.get_sparse_core_info()
    NC, NS = info.num_cores, info.num_subcores
    NW = NC * NS

    f = jnp.broadcast_to(distinction_table[None], (2,) + distinction_table.shape)
    s = jnp.broadcast_to(state_table[:, None, :], (2, num_kp, state_table.shape[-1]))
    fused = jnp.concatenate([f, s], axis=-1).reshape(2, _ROW)
    fused = jnp.tile(fused, (NW, 1))

    b_per_w = B // NW
    idx = grip_state.astype(jnp.int32).reshape(NW, b_per_w // _CH, _CH)
    idx = idx + 2 * jnp.arange(NW, dtype=jnp.int32)[:, None, None]

    out = _build_sc_call(B, NC, NS)(fused, idx)
    return out.reshape(B, num_kp, _ROW // num_kp)


# speedup vs baseline: 2.6901x; 1.0046x over previous
import functools

import jax
import jax.numpy as jnp
from jax import lax
from jax.experimental import pallas as pl
from jax.experimental.pallas import tpu as pltpu
from jax.experimental.pallas import tpu_sc as plsc

_ROW = 768
_CH = 64
_L = 16


def _build_sc_call(B, NC, NS, num_kp, d_dist, d_state):
    NW = NC * NS
    b_per_w = B // NW
    n_ch = b_per_w // _CH
    d_out = d_dist + d_state
    mesh = plsc.VectorSubcoreMesh(core_axis_name="c", subcore_axis_name="s")

    @functools.partial(
        pl.kernel,
        mesh=mesh,
        out_type=(
            jax.ShapeDtypeStruct((B, _ROW), jnp.float32),
            jax.ShapeDtypeStruct((NW * 2, _ROW), jnp.float32),
        ),
        scratch_types=[
            pltpu.VMEM((num_kp, d_dist), jnp.float32),
            pltpu.VMEM((2, d_state), jnp.float32),
            pltpu.VMEM((2, _ROW), jnp.float32),
            pltpu.VMEM((n_ch, _CH), jnp.int32),
            pltpu.VMEM((n_ch, _CH), jnp.int32),
            pltpu.VMEM((2, _CH, _ROW), jnp.float32),
            pltpu.SemaphoreType.DMA,
            pltpu.SemaphoreType.DMA,
            pltpu.SemaphoreType.DMA,
            pltpu.SemaphoreType.DMA,
        ],
    )
    def sc_gather(dist_hbm, state_hbm, idx_hbm, out_hbm, table_hbm,
                  dist_v, state_v, fused_v, idx_v, idx2_v, rows_v,
                  g0, g1, s0, s1):
        wid = lax.axis_index("s") * NC + lax.axis_index("c")
        base = wid * b_per_w

        pltpu.sync_copy(dist_hbm, dist_v)
        pltpu.sync_copy(state_hbm, state_v)
        pltpu.sync_copy(idx_hbm.at[wid], idx_v)

        for g in range(2):
            for k in range(num_kp):
                col = k * d_out
                for j in range(d_dist // _L):
                    fused_v[g, pl.ds(col + j * _L, _L)] = (
                        dist_v[k, pl.ds(j * _L, _L)])
                for j in range(d_state // _L):
                    fused_v[g, pl.ds(col + d_dist + j * _L, _L)] = (
                        state_v[g, pl.ds(j * _L, _L)])
        pltpu.sync_copy(fused_v, table_hbm.at[pl.ds(2 * wid, 2)])

        off = jnp.broadcast_to(2 * wid, (_L,)).astype(jnp.int32)
        for c in range(n_ch):
            for j in range(_CH // _L):
                idx2_v[c, pl.ds(j * _L, _L)] = (
                    idx_v[c, pl.ds(j * _L, _L)] + off)

        gsem = (g0, g1)
        ssem = (s0, s1)
        gat = [None, None]
        sca = [None, None]
        gat[0] = pltpu.async_copy(table_hbm.at[idx2_v.at[0]], rows_v.at[0],
                                  gsem[0])
        for c in range(n_ch):
            p = c & 1
            q = p ^ 1
            if c + 1 < n_ch:
                if sca[q] is not None:
                    sca[q].wait()
                gat[q] = pltpu.async_copy(
                    table_hbm.at[idx2_v.at[c + 1]], rows_v.at[q], gsem[q])
            gat[p].wait()
            sca[p] = pltpu.async_copy(
                rows_v.at[p], out_hbm.at[pl.ds(base + c * _CH, _CH)], ssem[p])
        for p in range(2):
            if sca[p] is not None:
                sca[p].wait()

    return sc_gather


def kernel(grip_state, distinction_table, state_table):
    B = grip_state.shape[0]
    num_kp, d_dist = distinction_table.shape
    d_state = state_table.shape[-1]
    info = plsc.get_sparse_core_info()
    NC, NS = info.num_cores, info.num_subcores
    NW = NC * NS

    b_per_w = B // NW
    idx = grip_state.astype(jnp.int32).reshape(NW, b_per_w // _CH, _CH)

    out, _ = _build_sc_call(B, NC, NS, num_kp, d_dist, d_state)(
        distinction_table, state_table, idx)
    return out.reshape(B, num_kp, d_dist + d_state)


# speedup vs baseline: 6.7223x; 2.4989x over previous
import functools

import jax
import jax.numpy as jnp
from jax.experimental import pallas as pl
from jax.experimental.pallas import tpu as pltpu

_BBLK = 1024


def _tc_body(gs_ref, dist_ref, state_ref, out_ref):
    dist = dist_ref[...]
    state = state_ref[...]
    nkp = dist.shape[0]
    p = jnp.concatenate([
        jnp.broadcast_to(dist[None], (2, nkp, dist.shape[-1])),
        jnp.broadcast_to(state[:, None, :], (2, nkp, state.shape[-1])),
    ], axis=-1)
    g = gs_ref[...]
    cond = (g == 0)[:, :, None]
    out_ref[...] = jnp.where(cond, p[0][None], p[1][None])


def kernel(grip_state, distinction_table, state_table):
    B = grip_state.shape[0]
    num_kp, d_dist = distinction_table.shape
    d_state = state_table.shape[-1]
    d_out = d_dist + d_state

    out = pl.pallas_call(
        _tc_body,
        out_shape=jax.ShapeDtypeStruct((B, num_kp, d_out), jnp.float32),
        grid=(B // _BBLK,),
        in_specs=[
            pl.BlockSpec((_BBLK, 1), lambda i: (i, 0)),
            pl.BlockSpec((num_kp, d_dist), lambda i: (0, 0)),
            pl.BlockSpec((2, d_state), lambda i: (0, 0)),
        ],
        out_specs=pl.BlockSpec((_BBLK, num_kp, d_out), lambda i: (i, 0, 0)),
    )(grip_state.astype(jnp.int32).reshape(B, 1), distinction_table,
      state_table)
    return out
